# Initial kernel scaffold; baseline (speedup 1.0000x reference)
import jax
import jax.numpy as jnp
from jax.experimental import pallas as pl


def kernel(x, edge_index, W_self0, W_neigh0, b0, W_self1, W_neigh1, b1):
    raise NotImplementedError("write your pallas kernel here")


# speedup vs baseline: 5.8013x; 5.8013x over previous
import functools

import jax
import jax.numpy as jnp
from jax import lax
from jax.experimental import pallas as pl
from jax.experimental.pallas import tpu as pltpu
from jax.experimental.pallas import tpu_sc as plsc

N_NODES = 10000
N_PAD = 10240
NC, NS = 2, 16
NW = NC * NS
EDGE_B = 128
DUMP_ROW = N_NODES + 64
ROW_BLK = 1024


def _tc_matmul(x, w):
    n, k = x.shape
    d = w.shape[1]

    def body(x_ref, w_ref, o_ref):
        o_ref[...] = jnp.dot(x_ref[...], w_ref[...],
                             preferred_element_type=jnp.float32)

    return pl.pallas_call(
        body,
        grid=(n // ROW_BLK,),
        in_specs=[
            pl.BlockSpec((ROW_BLK, k), lambda i: (i, 0)),
            pl.BlockSpec((k, d), lambda i: (0, 0)),
        ],
        out_specs=pl.BlockSpec((ROW_BLK, d), lambda i: (i, 0)),
        out_shape=jax.ShapeDtypeStruct((n, d), jnp.float32),
    )(x, w)


def _tc_combine(x, w_self, parts, b, relu, w_next=None):
    n, k = x.shape
    d = w_self.shape[1]
    d2 = None if w_next is None else w_next.shape[1]

    def body(x_ref, ws_ref, p_ref, b_ref, *rest):
        h = jnp.dot(x_ref[...], ws_ref[...], preferred_element_type=jnp.float32)
        h = h + p_ref[0] + p_ref[1] + b_ref[...]
        if relu:
            h = jnp.maximum(h, 0.0)
        if w_next is None:
            (o_ref,) = rest
            o_ref[...] = h
        else:
            wn_ref, h_ref, y_ref = rest
            h_ref[...] = h
            y_ref[...] = jnp.dot(h, wn_ref[...], preferred_element_type=jnp.float32)

    in_specs = [
        pl.BlockSpec((ROW_BLK, k), lambda i: (i, 0)),
        pl.BlockSpec((k, d), lambda i: (0, 0)),
        pl.BlockSpec((NC, ROW_BLK, d), lambda i: (0, i, 0)),
        pl.BlockSpec((1, d), lambda i: (0, 0)),
    ]
    operands = [x, w_self, parts, b.reshape(1, d)]
    if w_next is None:
        out_specs = pl.BlockSpec((ROW_BLK, d), lambda i: (i, 0))
        out_shape = jax.ShapeDtypeStruct((n, d), jnp.float32)
    else:
        in_specs.append(pl.BlockSpec((d, d2), lambda i: (0, 0)))
        operands.append(w_next)
        out_specs = [
            pl.BlockSpec((ROW_BLK, d), lambda i: (i, 0)),
            pl.BlockSpec((ROW_BLK, d2), lambda i: (i, 0)),
        ]
        out_shape = [
            jax.ShapeDtypeStruct((n, d), jnp.float32),
            jax.ShapeDtypeStruct((n, d2), jnp.float32),
        ]

    return pl.pallas_call(
        body,
        grid=(n // ROW_BLK,),
        in_specs=in_specs,
        out_specs=out_specs,
        out_shape=out_shape,
    )(*operands)


def _sc_agg(y, src3, dst3, zeros, n_j):
    d = y.shape[1]
    rows_per_s = N_PAD // NS
    mesh = plsc.VectorSubcoreMesh(core_axis_name="c", subcore_axis_name="s")

    @functools.partial(
        pl.kernel,
        out_type=jax.ShapeDtypeStruct((NC, N_PAD, d), jnp.float32),
        mesh=mesh,
        scratch_types=[
            pltpu.VMEM_SHARED((N_PAD, d), jnp.float32),
            pltpu.VMEM((n_j, EDGE_B), jnp.int32),
            pltpu.VMEM((n_j, EDGE_B), jnp.int32),
            pltpu.VMEM((EDGE_B, d), jnp.float32),
            pltpu.SemaphoreType.DMA,
        ],
        compiler_params=pltpu.CompilerParams(use_tc_tiling_on_sc=False),
    )
    def k(y_hbm, src_hbm, dst_hbm, z_hbm, out_hbm, acc, src_v, dst_v, rows_v, sem):
        c = lax.axis_index("c")
        s = lax.axis_index("s")
        w = s * NC + c
        r0 = s * rows_per_s
        pltpu.sync_copy(z_hbm.at[pl.ds(r0, rows_per_s)],
                        acc.at[pl.ds(r0, rows_per_s)])
        pltpu.sync_copy(src_hbm.at[w], src_v)
        pltpu.sync_copy(dst_hbm.at[w], dst_v)
        plsc.subcore_barrier()

        @pl.loop(0, n_j)
        def edge_chunk(j):
            pltpu.async_copy(y_hbm.at[src_v.at[j]], rows_v, sem).wait()
            pltpu.sync_copy(rows_v, acc.at[dst_v.at[j]], add=True)

        plsc.subcore_barrier()
        pltpu.sync_copy(acc.at[pl.ds(r0, rows_per_s)],
                        out_hbm.at[c, pl.ds(r0, rows_per_s)])

    return k(y, src3, dst3, zeros)


def kernel(x, edge_index, W_self0, W_neigh0, b0, W_self1, W_neigh1, b1):
    n_edges = edge_index.shape[1]
    per_w = -(-n_edges // (NW * EDGE_B)) * EDGE_B
    n_j = per_w // EDGE_B
    e_pad = NW * per_w

    x_pad = jnp.zeros((N_PAD, x.shape[1]), jnp.float32).at[:N_NODES].set(x)
    src = jnp.full((e_pad,), 0, jnp.int32).at[:n_edges].set(edge_index[0])
    dst = jnp.full((e_pad,), DUMP_ROW, jnp.int32).at[:n_edges].set(edge_index[1])
    src3 = src.reshape(NW, n_j, EDGE_B)
    dst3 = dst.reshape(NW, n_j, EDGE_B)

    d_hid = W_neigh0.shape[1]
    d_out = W_neigh1.shape[1]
    zeros_hid = jnp.zeros((N_PAD, d_hid), jnp.float32)
    zeros_out = jnp.zeros((N_PAD, d_out), jnp.float32)

    y0 = _tc_matmul(x_pad, W_neigh0)
    p0 = _sc_agg(y0, src3, dst3, zeros_hid, n_j)
    h, y1 = _tc_combine(x_pad, W_self0, p0, b0, relu=True, w_next=W_neigh1)
    p1 = _sc_agg(y1, src3, dst3, zeros_out, n_j)
    out = _tc_combine(h, W_self1, p1, b1, relu=False)
    return out[:N_NODES]
